# Initial kernel scaffold; baseline (speedup 1.0000x reference)
import jax
import jax.numpy as jnp
from jax import lax
from jax.experimental import pallas as pl
from jax.experimental.pallas import tpu as pltpu
from jax.experimental.pallas import tpu_sc as plsc

B, T, V, D = 1024, 802, 133, 32
TP = 832
NCHUNK, CW = 8, 104
NC, NS = 2, 16
NW = NC * NS
ROWS_PER_W = B // NW


def _prep_body(xp_ref, w_ref, pe_ref, fused_ref, idx_ref):
    fused_ref[...] = pe_ref[...][:, None, :] + w_ref[...][None, :, :]
    t = lax.broadcasted_iota(jnp.int32, (B, TP), 1)
    idx_ref[...] = jnp.where(t < T, xp_ref[...] + t * V, 0)


def _prep(xp, w, pe2, interpret=False):
    return pl.pallas_call(
        _prep_body,
        out_shape=(
            jax.ShapeDtypeStruct((T, V, D), jnp.float32),
            jax.ShapeDtypeStruct((B, TP), jnp.int32),
        ),
        interpret=interpret,
    )(xp, w, pe2)


def _sc_gather(fused_flat, idx3):
    mesh = plsc.VectorSubcoreMesh(core_axis_name="c", subcore_axis_name="s")

    @pl.kernel(
        out_type=jax.ShapeDtypeStruct((B, T, D), jnp.float32),
        mesh=mesh,
        scratch_types=[
            pltpu.VMEM((NCHUNK, CW), jnp.int32),
            pltpu.VMEM((TP, D), jnp.float32),
            pltpu.SemaphoreType.DMA,
        ],
    )
    def k(fused_hbm, idx_hbm, out_hbm, idx_v, rows_v, sem):
        wid = lax.axis_index("s") * NC + lax.axis_index("c")

        @pl.loop(0, ROWS_PER_W)
        def _(i):
            b = wid * ROWS_PER_W + i
            pltpu.sync_copy(idx_hbm.at[b], idx_v)
            copies = [
                pltpu.async_copy(
                    fused_hbm.at[idx_v.at[c]],
                    rows_v.at[pl.ds(c * CW, CW)],
                    sem,
                )
                for c in range(NCHUNK)
            ]
            for cp in copies:
                cp.wait()
            pltpu.sync_copy(rows_v.at[pl.ds(0, T)], out_hbm.at[b])

    return k(fused_flat, idx3)


def kernel(x, embed_weight, pe):
    x32 = x.astype(jnp.int32)
    xp = jnp.pad(x32, ((0, 0), (0, TP - T)))
    pe2 = pe.reshape(T, D)
    fused, idx = _prep(xp, embed_weight, pe2)
    return _sc_gather(fused.reshape(T * V, D), idx.reshape(B, NCHUNK, CW))


# speedup vs baseline: 2.8346x; 2.8346x over previous
import jax
import jax.numpy as jnp
from jax import lax
from jax.experimental import pallas as pl
from jax.experimental.pallas import tpu as pltpu
from jax.experimental.pallas import tpu_sc as plsc

B, T, V, D = 1024, 802, 133, 32
TP = 832
NCHUNK, CW = 8, 104
NC, NS = 2, 16
NW = NC * NS
ROWS_PER_W = B // NW


def _prep_body(xp_ref, wflat_ref, pe_ref, fused_ref, idx_ref):
    pe_tiled = jnp.concatenate([pe_ref[...]] * V, axis=1)
    fused_ref[...] = pe_tiled + wflat_ref[...]
    t = lax.broadcasted_iota(jnp.int32, (B, TP), 1)
    idx_ref[...] = jnp.where(t < T, xp_ref[...] + t * V, 0)


def _prep(xp, wflat, pe2, interpret=False):
    return pl.pallas_call(
        _prep_body,
        out_shape=(
            jax.ShapeDtypeStruct((T, V * D), jnp.float32),
            jax.ShapeDtypeStruct((B, TP), jnp.int32),
        ),
        interpret=interpret,
    )(xp, wflat, pe2)


def _sc_gather(fused_flat, idx3):
    mesh = plsc.VectorSubcoreMesh(core_axis_name="c", subcore_axis_name="s")

    @pl.kernel(
        out_type=jax.ShapeDtypeStruct((B, T, D), jnp.float32),
        mesh=mesh,
        compiler_params=pltpu.CompilerParams(use_tc_tiling_on_sc=False),
        scratch_types=[
            pltpu.VMEM((NCHUNK, CW), jnp.int32),
            pltpu.VMEM((TP, D), jnp.float32),
            pltpu.SemaphoreType.DMA,
        ],
    )
    def k(fused_hbm, idx_hbm, out_hbm, idx_v, rows_v, sem):
        wid = lax.axis_index("s") * NC + lax.axis_index("c")

        @pl.loop(0, ROWS_PER_W)
        def _(i):
            b = wid * ROWS_PER_W + i
            pltpu.sync_copy(idx_hbm.at[b], idx_v)
            copies = [
                pltpu.async_copy(
                    fused_hbm.at[idx_v.at[c]],
                    rows_v.at[pl.ds(c * CW, CW)],
                    sem,
                )
                for c in range(NCHUNK)
            ]
            for cp in copies:
                cp.wait()
            pltpu.sync_copy(rows_v.at[pl.ds(0, T)], out_hbm.at[b])

    return k(fused_flat, idx3)


def kernel(x, embed_weight, pe):
    x32 = x.astype(jnp.int32)
    xp = jnp.pad(x32, ((0, 0), (0, TP - T)))
    pe2 = pe.reshape(T, D)
    wflat = embed_weight.reshape(1, V * D)
    fused, idx = _prep(xp, wflat, pe2)
    return _sc_gather(fused.reshape(T * V, D), idx.reshape(B, NCHUNK, CW))


# speedup vs baseline: 4.0744x; 1.4374x over previous
import jax
import jax.numpy as jnp
from jax import lax
from jax.experimental import pallas as pl
from jax.experimental.pallas import tpu as pltpu
from jax.experimental.pallas import tpu_sc as plsc

B, T, V, D = 1024, 802, 133, 32
TP = 832
NC, NS = 2, 16
NW = NC * NS
TW = TP // NW
ROW = 4 * 8 * 8 * 128


def _sc_lookup(wflat, idxt, pe2):
    mesh = plsc.VectorSubcoreMesh(core_axis_name="c", subcore_axis_name="s")

    @pl.kernel(
        out_type=jax.ShapeDtypeStruct((T, ROW), jnp.float32),
        mesh=mesh,
        compiler_params=pltpu.CompilerParams(
            use_tc_tiling_on_sc=False, needs_layout_passes=False
        ),
        scratch_types=[
            pltpu.VMEM((V * D,), jnp.float32),
            pltpu.VMEM((B,), jnp.int32),
            pltpu.VMEM((ROW,), jnp.float32),
            pltpu.VMEM((D,), jnp.float32),
        ],
    )
    def k(w_hbm, idxt_hbm, pe_hbm, out_hbm, w_v, xcol_v, outv, pe_row):
        wid = lax.axis_index("s") * NC + lax.axis_index("c")
        pltpu.sync_copy(w_hbm, w_v)

        @pl.loop(0, TW)
        def _(kk):
            t = wid * TW + kk

            @pl.when(t < T)
            def _():
                pltpu.sync_copy(idxt_hbm.at[t], xcol_v)
                pltpu.sync_copy(pe_hbm.at[t], pe_row)
                pe_lo = pe_row[pl.ds(0, 16)]
                pe_hi = pe_row[pl.ds(16, 16)]
                pes = [pe_lo[d] for d in range(16)] + [pe_hi[d] for d in range(16)]

                @pl.loop(0, 8)
                def _(bh):
                    for q in range(8):
                        xv = xcol_v[pl.ds(bh * 128 + q * 16, 16)] * D
                        for d in range(D):
                            g = plsc.load_gather(w_v, [xv + d])
                            off = (d // 8) * 8192 + (d % 8) * 128 + q * 16
                            outv[pl.ds(bh * 1024 + off, 16)] = g + pes[d]

                pltpu.sync_copy(outv, out_hbm.at[t])

    return k(wflat, idxt, pe2)


def kernel(x, embed_weight, pe):
    x32 = x.astype(jnp.int32)
    idxt = jnp.transpose(x32)
    wflat = embed_weight.reshape(V * D)
    pe2 = pe.reshape(T, D)
    res = _sc_lookup(wflat, idxt, pe2)
    out4 = res.reshape(T, 4, 8, 8, 128)
    return out4.transpose(2, 4, 0, 1, 3).reshape(B, T, D)


# speedup vs baseline: 8.5168x; 2.0903x over previous
import jax
import jax.numpy as jnp
from jax import lax
from jax.experimental import pallas as pl
from jax.experimental.pallas import tpu as pltpu
from jax.experimental.pallas import tpu_sc as plsc

B, T, V, D = 1024, 802, 133, 32
TP = 832
NC, NS = 2, 16
NW = NC * NS
TW = TP // NW
ROW = 4 * 8 * 8 * 128


def _sc_lookup(wflat, idxt, pe2):
    mesh = plsc.VectorSubcoreMesh(core_axis_name="c", subcore_axis_name="s")

    @pl.kernel(
        out_type=jax.ShapeDtypeStruct((T, ROW), jnp.float32),
        mesh=mesh,
        compiler_params=pltpu.CompilerParams(
            use_tc_tiling_on_sc=False, needs_layout_passes=False
        ),
        scratch_types=[
            pltpu.VMEM((V * D,), jnp.float32),
            pltpu.VMEM((B,), jnp.int32),
            pltpu.VMEM((ROW,), jnp.float32),
            pltpu.VMEM((D,), jnp.float32),
        ],
    )
    def k(w_hbm, idxt_hbm, pe_hbm, out_hbm, w_v, xcol_v, outv, pe_row):
        wid = lax.axis_index("s") * NC + lax.axis_index("c")
        pltpu.sync_copy(w_hbm, w_v)

        @pl.loop(0, TW)
        def _(kk):
            t = wid * TW + kk

            @pl.when(t < T)
            def _():
                pltpu.sync_copy(idxt_hbm.at[t], xcol_v)
                pltpu.sync_copy(pe_hbm.at[t], pe_row)
                pe_lo = pe_row[pl.ds(0, 16)]
                pe_hi = pe_row[pl.ds(16, 16)]
                pes = [pe_lo[d] for d in range(16)] + [pe_hi[d] for d in range(16)]

                @pl.loop(0, 8)
                def _(bh):
                    for q in range(8):
                        xv = xcol_v[pl.ds(bh * 128 + q * 16, 16)]
                        for d in range(D):
                            g = plsc.load_gather(w_v, [xv + d * V])
                            off = (d // 8) * 8192 + (d % 8) * 128 + q * 16
                            outv[pl.ds(bh * 1024 + off, 16)] = g + pes[d]

                pltpu.sync_copy(outv, out_hbm.at[t])

    return k(wflat, idxt, pe2)


def kernel(x, embed_weight, pe):
    x32 = x.astype(jnp.int32)
    idxt = jnp.transpose(x32)
    wflat = embed_weight.T.reshape(D * V)
    pe2 = pe.reshape(T, D)
    res = _sc_lookup(wflat, idxt, pe2)
    out4 = res.reshape(T, 4, 8, 8, 128)
    return out4.transpose(2, 4, 0, 1, 3).reshape(B, T, D)


# speedup vs baseline: 14.0931x; 1.6547x over previous
import jax
import jax.numpy as jnp
from jax import lax
from jax.experimental import pallas as pl
from jax.experimental.pallas import tpu as pltpu
from jax.experimental.pallas import tpu_sc as plsc

B, T, V, D = 1024, 802, 133, 32
TP = 832
NC, NS = 2, 16
NW = NC * NS
TW = TP // NW
ROW = 4 * 8 * 8 * 128


def _sc_lookup(wflat, idxt, pe2):
    mesh = plsc.VectorSubcoreMesh(core_axis_name="c", subcore_axis_name="s")

    @pl.kernel(
        out_type=jax.ShapeDtypeStruct((T, ROW), jnp.float32),
        mesh=mesh,
        compiler_params=pltpu.CompilerParams(
            use_tc_tiling_on_sc=False, needs_layout_passes=False
        ),
        scratch_types=[
            pltpu.VMEM((V * D * 16,), jnp.float32),
            pltpu.VMEM((V * D,), jnp.float32),
            pltpu.VMEM((B,), jnp.int32),
            pltpu.VMEM((ROW,), jnp.float32),
            pltpu.VMEM((D,), jnp.float32),
        ],
    )
    def k(w_hbm, idxt_hbm, pe_hbm, out_hbm, rep_v, w_v, xcol_v, outv, pe_row):
        wid = lax.axis_index("s") * NC + lax.axis_index("c")
        lane = lax.iota(jnp.int32, 16)
        pltpu.sync_copy(w_hbm, w_v)

        @plsc.parallel_loop(0, V * D // 16, unroll=2)
        def _(i):
            wv = w_v[pl.ds(i * 16, 16)]
            for j in range(16):
                rep_v[pl.ds((i * 16 + j) * 16, 16)] = jnp.broadcast_to(wv[j], (16,))

        @pl.loop(0, TW)
        def _(kk):
            t = wid * TW + kk

            @pl.when(t < T)
            def _():
                pltpu.sync_copy(idxt_hbm.at[t], xcol_v)
                pltpu.sync_copy(pe_hbm.at[t], pe_row)
                pe_lo = pe_row[pl.ds(0, 16)]
                pe_hi = pe_row[pl.ds(16, 16)]
                pes = [pe_lo[d] for d in range(16)] + [pe_hi[d] for d in range(16)]

                @plsc.parallel_loop(0, 64, unroll=2)
                def _(c):
                    xv16 = (xcol_v[pl.ds(c * 16, 16)] << 4) + lane
                    coff = (c // 8) * 1024 + (c % 8) * 16
                    for d in range(D):
                        g = plsc.load_gather(rep_v, [xv16 + d * V * 16])
                        off = (d // 8) * 8192 + (d % 8) * 128
                        outv[pl.ds(coff + off, 16)] = g + pes[d]

                pltpu.sync_copy(outv, out_hbm.at[t])

    return k(wflat, idxt, pe2)


def kernel(x, embed_weight, pe):
    x32 = x.astype(jnp.int32)
    idxt = jnp.transpose(x32)
    wflat = embed_weight.T.reshape(D * V)
    pe2 = pe.reshape(T, D)
    res = _sc_lookup(wflat, idxt, pe2)
    out4 = res.reshape(T, 4, 8, 8, 128)
    return out4.transpose(2, 4, 0, 1, 3).reshape(B, T, D)


# speedup vs baseline: 26.7193x; 1.8959x over previous
import jax
import jax.numpy as jnp
from jax import lax
from jax.experimental import pallas as pl
from jax.experimental.pallas import tpu as pltpu
from jax.experimental.pallas import tpu_sc as plsc

B, T, V, D = 1024, 802, 133, 32
TP = 832
NC, NS = 2, 16
NW = NC * NS
TW = TP // NW
ROW = 4 * 8 * 8 * 128


def _sc_lookup(wflat, idxt, pe2):
    mesh = plsc.VectorSubcoreMesh(core_axis_name="c", subcore_axis_name="s")

    @pl.kernel(
        out_type=jax.ShapeDtypeStruct((T, ROW), jnp.float32),
        mesh=mesh,
        compiler_params=pltpu.CompilerParams(
            use_tc_tiling_on_sc=False, needs_layout_passes=False
        ),
        scratch_types=[
            pltpu.VMEM((V * D * 16,), jnp.float32),
            pltpu.VMEM((V * D,), jnp.float32),
            pltpu.VMEM((B,), jnp.int32),
            pltpu.VMEM((ROW,), jnp.float32),
            pltpu.VMEM((D,), jnp.float32),
            pltpu.SemaphoreType.DMA,
            pltpu.SemaphoreType.DMA,
        ],
    )
    def k(w_hbm, idxt_hbm, pe_hbm, out_hbm, rep_v, w_v, xcol_v, outv, pe_row,
          osem0, osem1):
        wid = lax.axis_index("s") * NC + lax.axis_index("c")
        lane = lax.iota(jnp.int32, 16)
        pltpu.sync_copy(w_hbm, w_v)

        @plsc.parallel_loop(0, V * D // 16, unroll=2)
        def _(i):
            wv = w_v[pl.ds(i * 16, 16)]
            for j in range(16):
                rep_v[pl.ds((i * 16 + j) * 16, 16)] = jnp.broadcast_to(wv[j], (16,))

        HALF = ROW // 2
        osems = (osem0, osem1)

        def _wait_out(h):
            pltpu.make_async_copy(
                outv.at[pl.ds(0, HALF)], out_hbm.at[0, pl.ds(0, HALF)], osems[h]
            ).wait()

        @pl.loop(0, TW)
        def _(kk):
            t = wid * TW + kk

            @pl.when(t < T)
            def _():
                pltpu.sync_copy(idxt_hbm.at[t], xcol_v)
                pltpu.sync_copy(pe_hbm.at[t], pe_row)
                pe_lo = pe_row[pl.ds(0, 16)]
                pe_hi = pe_row[pl.ds(16, 16)]
                pes = [jnp.broadcast_to(pe_lo[d], (16,)) for d in range(16)]
                pes += [jnp.broadcast_to(pe_hi[d], (16,)) for d in range(16)]

                for h in range(2):
                    @pl.when(kk > 0)
                    def _():
                        _wait_out(h)

                    @plsc.parallel_loop(0, 64, unroll=4)
                    def _(c):
                        xv16 = (xcol_v[pl.ds(c * 16, 16)] << 4) + lane
                        coff = (c // 8) * 1024 + (c % 8) * 16
                        for d in range(h * 16, h * 16 + 16):
                            g = plsc.load_gather(rep_v, [xv16 + d * V * 16])
                            off = ((d % 16) // 8) * 8192 + (d % 8) * 128
                            outv[pl.ds(h * HALF + coff + off, 16)] = g + pes[d]

                    pltpu.async_copy(
                        outv.at[pl.ds(h * HALF, HALF)],
                        out_hbm.at[t, pl.ds(h * HALF, HALF)],
                        osems[h],
                    )

        @pl.when(wid * TW < T)
        def _():
            _wait_out(0)
            _wait_out(1)

    return k(wflat, idxt, pe2)


def kernel(x, embed_weight, pe):
    x32 = x.astype(jnp.int32)
    idxt = jnp.transpose(x32)
    wflat = embed_weight.T.reshape(D * V)
    pe2 = pe.reshape(T, D)
    res = _sc_lookup(wflat, idxt, pe2)
    out4 = res.reshape(T, 4, 8, 8, 128)
    return out4.transpose(2, 4, 0, 1, 3).reshape(B, T, D)


# speedup vs baseline: 31.9630x; 1.1963x over previous
import jax
import jax.numpy as jnp
from jax import lax
from jax.experimental import pallas as pl
from jax.experimental.pallas import tpu as pltpu
from jax.experimental.pallas import tpu_sc as plsc

B, T, V, D = 1024, 802, 133, 32
TP = 832
NC, NS = 2, 16
NW = NC * NS
TW = TP // NW
ROW = 4 * 8 * 8 * 128


def _sc_lookup(wflat, idxt, pe2):
    mesh = plsc.VectorSubcoreMesh(core_axis_name="c", subcore_axis_name="s")

    @pl.kernel(
        out_type=jax.ShapeDtypeStruct((T, ROW), jnp.float32),
        mesh=mesh,
        compiler_params=pltpu.CompilerParams(
            use_tc_tiling_on_sc=False, needs_layout_passes=False
        ),
        scratch_types=[
            pltpu.VMEM((V * D * 16,), jnp.float32),
            pltpu.VMEM((V * D,), jnp.float32),
            pltpu.VMEM((B,), jnp.int32),
            pltpu.VMEM((ROW,), jnp.float32),
            pltpu.VMEM((D,), jnp.float32),
            pltpu.SemaphoreType.DMA,
            pltpu.SemaphoreType.DMA,
        ],
    )
    def k(w_hbm, idxt_hbm, pe_hbm, out_hbm, rep_v, w_v, xcol_v, outv, pe_row,
          osem0, osem1):
        wid = lax.axis_index("s") * NC + lax.axis_index("c")
        lane = lax.iota(jnp.int32, 16)
        pltpu.sync_copy(w_hbm, w_v)

        @plsc.parallel_loop(0, V * D // 16, unroll=2)
        def _(i):
            wv = w_v[pl.ds(i * 16, 16)]
            for j in range(16):
                rep_v[pl.ds((i * 16 + j) * 16, 16)] = jnp.broadcast_to(wv[j], (16,))

        HALF = ROW // 2
        osems = (osem0, osem1)

        def _wait_out(h):
            pltpu.make_async_copy(
                outv.at[pl.ds(0, HALF)], out_hbm.at[0, pl.ds(0, HALF)], osems[h]
            ).wait()

        @pl.loop(0, TW)
        def _(kk):
            t = wid * TW + kk

            @pl.when(t < T)
            def _():
                pltpu.sync_copy(idxt_hbm.at[t], xcol_v)
                pltpu.sync_copy(pe_hbm.at[t], pe_row)
                pe_lo = pe_row[pl.ds(0, 16)]
                pe_hi = pe_row[pl.ds(16, 16)]
                pes = [pe_lo[d] for d in range(16)] + [pe_hi[d] for d in range(16)]

                for h in range(2):
                    @pl.when(kk > 0)
                    def _():
                        _wait_out(h)

                    @plsc.parallel_loop(0, 64, unroll=4)
                    def _(c):
                        xv16 = (xcol_v[pl.ds(c * 16, 16)] << 4) + lane
                        coff = (c // 8) * 1024 + (c % 8) * 16
                        for d in range(h * 16, h * 16 + 16):
                            g = plsc.load_gather(
                                rep_v.at[pl.ds(d * V * 16, V * 16)], [xv16]
                            )
                            off = ((d % 16) // 8) * 8192 + (d % 8) * 128
                            outv[pl.ds(h * HALF + coff + off, 16)] = g + pes[d]

                    pltpu.async_copy(
                        outv.at[pl.ds(h * HALF, HALF)],
                        out_hbm.at[t, pl.ds(h * HALF, HALF)],
                        osems[h],
                    )

        @pl.when(wid * TW < T)
        def _():
            _wait_out(0)
            _wait_out(1)

    return k(wflat, idxt, pe2)


def kernel(x, embed_weight, pe):
    x32 = x.astype(jnp.int32)
    idxt = jnp.transpose(x32)
    wflat = embed_weight.T.reshape(D * V)
    pe2 = pe.reshape(T, D)
    res = _sc_lookup(wflat, idxt, pe2)
    out4 = res.reshape(T, 4, 8, 8, 128)
    return out4.transpose(2, 4, 0, 1, 3).reshape(B, T, D)


# speedup vs baseline: 47.9283x; 1.4995x over previous
import jax
import jax.numpy as jnp
from jax import lax
from jax.experimental import pallas as pl
from jax.experimental.pallas import tpu as pltpu
from jax.experimental.pallas import tpu_sc as plsc

B, T, V, D = 1024, 802, 133, 32
TP = 832
NC, NS = 2, 16
NW = NC * NS
TW = TP // NW
ROW = 4 * 8 * 8 * 128


def _sc_lookup(wflat, idxt, pe2):
    mesh = plsc.VectorSubcoreMesh(core_axis_name="c", subcore_axis_name="s")

    @pl.kernel(
        out_type=jax.ShapeDtypeStruct((T, ROW), jnp.float32),
        mesh=mesh,
        compiler_params=pltpu.CompilerParams(
            use_tc_tiling_on_sc=False, needs_layout_passes=False
        ),
        scratch_types=[
            pltpu.VMEM((V * D * 16,), jnp.float32),
            pltpu.VMEM((V * D,), jnp.float32),
            pltpu.VMEM((2, B), jnp.int32),
            pltpu.VMEM((ROW,), jnp.float32),
            pltpu.VMEM((2, D), jnp.float32),
            pltpu.SemaphoreType.DMA,
            pltpu.SemaphoreType.DMA,
            pltpu.SemaphoreType.DMA,
            pltpu.SemaphoreType.DMA,
        ],
    )
    def k(w_hbm, idxt_hbm, pe_hbm, out_hbm, rep_v, w_v, xcol_v, outv, pe_row,
          osem0, osem1, xsem, psem):
        wid = lax.axis_index("s") * NC + lax.axis_index("c")
        lane = lax.iota(jnp.int32, 16)
        pltpu.sync_copy(w_hbm, w_v)

        @plsc.parallel_loop(0, V * D // 16, unroll=2)
        def _(i):
            wv = w_v[pl.ds(i * 16, 16)]
            for j in range(16):
                rep_v[pl.ds((i * 16 + j) * 16, 16)] = jnp.broadcast_to(wv[j], (16,))

        HALF = ROW // 2
        osems = (osem0, osem1)

        def _wait_out(h):
            pltpu.make_async_copy(
                outv.at[pl.ds(0, HALF)], out_hbm.at[0, pl.ds(0, HALF)], osems[h]
            ).wait()

        def _fetch_in(t, slot):
            pltpu.async_copy(idxt_hbm.at[t], xcol_v.at[slot], xsem)
            pltpu.async_copy(pe_hbm.at[t], pe_row.at[slot], psem)

        def _wait_in(slot):
            pltpu.make_async_copy(pe_hbm.at[0], pe_row.at[slot], psem).wait()
            pltpu.make_async_copy(idxt_hbm.at[0], xcol_v.at[slot], xsem).wait()

        t0 = wid * TW

        @pl.when(t0 < T)
        def _():
            _fetch_in(t0, 0)

        @pl.loop(0, TW)
        def _(kk):
            t = t0 + kk

            @pl.when(t < T)
            def _():
                cur = kk % 2

                @pl.when((kk + 1 < TW) & (t + 1 < T))
                def _():
                    _fetch_in(t + 1, 1 - cur)

                _wait_in(cur)
                pe_lo = pe_row[cur, pl.ds(0, 16)]
                pe_hi = pe_row[cur, pl.ds(16, 16)]
                pes = [pe_lo[d] for d in range(16)] + [pe_hi[d] for d in range(16)]

                for h in range(2):
                    @pl.when(kk > 0)
                    def _():
                        _wait_out(h)

                    @plsc.parallel_loop(0, 64, unroll=4)
                    def _(c):
                        xv16 = (xcol_v[cur, pl.ds(c * 16, 16)] << 4) + lane
                        coff = (c // 8) * 1024 + (c % 8) * 16
                        for d in range(h * 16, h * 16 + 16):
                            g = plsc.load_gather(
                                rep_v.at[pl.ds(d * V * 16, V * 16)], [xv16]
                            )
                            off = ((d % 16) // 8) * 8192 + (d % 8) * 128
                            outv[pl.ds(h * HALF + coff + off, 16)] = g + pes[d]

                    pltpu.async_copy(
                        outv.at[pl.ds(h * HALF, HALF)],
                        out_hbm.at[t, pl.ds(h * HALF, HALF)],
                        osems[h],
                    )

        @pl.when(wid * TW < T)
        def _():
            _wait_out(0)
            _wait_out(1)

    return k(wflat, idxt, pe2)


def kernel(x, embed_weight, pe):
    x32 = x.astype(jnp.int32)
    idxt = jnp.transpose(x32)
    wflat = embed_weight.T.reshape(D * V)
    pe2 = pe.reshape(T, D)
    res = _sc_lookup(wflat, idxt, pe2)
    out4 = res.reshape(T, 4, 8, 8, 128)
    return out4.transpose(2, 4, 0, 1, 3).reshape(B, T, D)
